# baseline (speedup 1.0000x reference)
import functools

import jax
import jax.numpy as jnp
from jax import lax
from jax.experimental import pallas as pl
from jax.experimental.pallas import tpu as pltpu
from jax.experimental.pallas import tpu_sc as plsc


def _make_sc_gather(NB, T, D):
    info = plsc.get_sparse_core_info()
    nc, ns, L = info.num_cores, info.num_subcores, info.num_lanes
    nw = nc * ns
    assert NB % nw == 0
    sn = NB // nw
    assert sn % L == 0 and T % 2 == 0
    mesh = plsc.VectorSubcoreMesh(core_axis_name="c", subcore_axis_name="s")

    @functools.partial(
        pl.kernel,
        mesh=mesh,
        compiler_params=pltpu.CompilerParams(
            use_tc_tiling_on_sc=False, needs_layout_passes=False
        ),
        out_type=jax.ShapeDtypeStruct((NB, T, D), jnp.float32),
        scratch_types=[
            pltpu.VMEM((sn, T), jnp.int32),
            pltpu.VMEM((sn * T,), jnp.int32),
            pltpu.VMEM((sn, D), jnp.float32),
            pltpu.VMEM((sn, D), jnp.float32),
            pltpu.SemaphoreType.DMA,
            pltpu.SemaphoreType.DMA,
            pltpu.SemaphoreType.DMA,
            pltpu.SemaphoreType.DMA,
        ],
    )
    def gather_kernel(
        tok_hbm, table_hbm, out_hbm, stage_v, idx_v, rows0, rows1, g0, g1, w0, w1
    ):
        wid = lax.axis_index("s") * nc + lax.axis_index("c")
        sbase = wid * sn
        pltpu.sync_copy(tok_hbm.at[pl.ds(sbase, sn)], stage_v)

        lanes = lax.iota(jnp.int32, L)

        def transpose_t(t, carry):
            def block(k, carry):
                s0 = k * L
                vals = plsc.load_gather(
                    stage_v, [s0 + lanes, jnp.full((L,), t, jnp.int32)]
                )
                idx_v[pl.ds(t * sn + s0, L)] = vals
                return carry

            return lax.fori_loop(0, sn // L, block, carry)

        lax.fori_loop(0, T, transpose_t, 0)

        def gather(t, buf, sem):
            return pltpu.make_async_copy(
                table_hbm.at[idx_v.at[pl.ds(t * sn, sn)]], buf, sem
            )

        def write(t, buf, sem):
            return pltpu.make_async_copy(
                buf, out_hbm.at[pl.ds(sbase, sn), t], sem
            )

        gather(0, rows0, g0).start()
        gather(1, rows1, g1).start()
        for g in range(T // 2):
            for b, (buf, gsem, wsem) in enumerate(
                ((rows0, g0, w0), (rows1, g1, w1))
            ):
                t = g * 2 + b
                gather(t, buf, gsem).wait()
                write(t, buf, wsem).start()
                write(t, buf, wsem).wait()
                if t + 2 < T:
                    gather(t + 2, buf, gsem).start()

    return gather_kernel


def kernel(tokens, table):
    nb, t = tokens.shape
    d = table.shape[1]
    return _make_sc_gather(nb, t, d)(tokens, table)


# speedup vs baseline: 1.8860x; 1.8860x over previous
import functools

import jax
import jax.numpy as jnp
from jax import lax
from jax.experimental import pallas as pl
from jax.experimental.pallas import tpu as pltpu
from jax.experimental.pallas import tpu_sc as plsc


def _make_sc_gather(NB, T, D, TP, DP):
    info = plsc.get_sparse_core_info()
    nc, ns = info.num_cores, info.num_subcores
    nw = nc * ns
    assert NB % nw == 0
    sn = NB // nw
    assert T % 2 == 0
    mesh = plsc.VectorSubcoreMesh(core_axis_name="c", subcore_axis_name="s")

    @functools.partial(
        pl.kernel,
        mesh=mesh,
        compiler_params=pltpu.CompilerParams(
            use_tc_tiling_on_sc=False, needs_layout_passes=False
        ),
        out_type=jax.ShapeDtypeStruct((NB, TP, DP), jnp.float32),
        scratch_types=[
            pltpu.VMEM((T, sn), jnp.int32),
            pltpu.VMEM((sn, D), jnp.float32),
            pltpu.VMEM((sn, D), jnp.float32),
            pltpu.SemaphoreType.DMA,
            pltpu.SemaphoreType.DMA,
            pltpu.SemaphoreType.DMA,
            pltpu.SemaphoreType.DMA,
            pltpu.SemaphoreType.DMA,
        ],
    )
    def gather_kernel(
        tok_hbm, table_hbm, out_hbm, idx_v, rows0, rows1, si, g0, g1, w0, w1
    ):
        wid = lax.axis_index("s") * nc + lax.axis_index("c")
        sbase = wid * sn

        def idx_load(t):
            return pltpu.make_async_copy(
                tok_hbm.at[pl.ds(t * NB + sbase, sn)], idx_v.at[t], si
            )

        for t in range(T):
            idx_load(t).start()
        for t in range(T):
            idx_load(t).wait()

        def gather(t, buf, sem):
            return pltpu.make_async_copy(
                table_hbm.at[idx_v.at[t]], buf, sem
            )

        def write(t, buf, sem):
            return pltpu.make_async_copy(
                buf, out_hbm.at[pl.ds(sbase, sn), t, pl.ds(0, D)], sem
            )

        gather(0, rows0, g0).start()
        gather(1, rows1, g1).start()
        for g in range(T // 2):
            for b, (buf, gsem, wsem) in enumerate(
                ((rows0, g0, w0), (rows1, g1, w1))
            ):
                t = g * 2 + b
                gather(t, buf, gsem).wait()
                write(t, buf, wsem).start()
                write(t, buf, wsem).wait()
                if t + 2 < T:
                    gather(t + 2, buf, gsem).start()

    return gather_kernel


def kernel(tokens, table):
    nb, t = tokens.shape
    d = table.shape[1]
    tp = (t + 7) // 8 * 8
    dp = 128
    idx_tmajor = tokens.T.reshape(nb * t)
    out_padded = _make_sc_gather(nb, t, d, tp, dp)(idx_tmajor, table)
    return out_padded[:, :t, :d]


# speedup vs baseline: 1.8989x; 1.0068x over previous
import functools

import jax
import jax.numpy as jnp
from jax import lax
from jax.experimental import pallas as pl
from jax.experimental.pallas import tpu as pltpu
from jax.experimental.pallas import tpu_sc as plsc


def _make_sc_gather(NB, T, D, TP, DP):
    info = plsc.get_sparse_core_info()
    nc, ns = info.num_cores, info.num_subcores
    nw = nc * ns
    assert NB % nw == 0
    sn = NB // nw
    assert T % 4 == 0
    mesh = plsc.VectorSubcoreMesh(core_axis_name="c", subcore_axis_name="s")

    @functools.partial(
        pl.kernel,
        mesh=mesh,
        compiler_params=pltpu.CompilerParams(
            use_tc_tiling_on_sc=False, needs_layout_passes=False
        ),
        out_type=jax.ShapeDtypeStruct((NB, TP, DP), jnp.float32),
        scratch_types=[
            pltpu.VMEM((T * sn,), jnp.int32),
            pltpu.VMEM((2 * sn, D), jnp.float32),
            pltpu.VMEM((2 * sn, D), jnp.float32),
            pltpu.SemaphoreType.DMA,
            pltpu.SemaphoreType.DMA,
            pltpu.SemaphoreType.DMA,
            pltpu.SemaphoreType.DMA,
            pltpu.SemaphoreType.DMA,
        ],
    )
    def gather_kernel(
        tok_hbm, table_hbm, out_hbm, idx_v, rows0, rows1, si, g0, g1, w0, w1
    ):
        wid = lax.axis_index("s") * nc + lax.axis_index("c")
        sbase = wid * sn

        def idx_load(t):
            return pltpu.make_async_copy(
                tok_hbm.at[pl.ds(t * NB + sbase, sn)],
                idx_v.at[pl.ds(t * sn, sn)],
                si,
            )

        for t in range(T):
            idx_load(t).start()
        for t in range(T):
            idx_load(t).wait()

        def gather(r, buf, sem):
            return pltpu.make_async_copy(
                table_hbm.at[idx_v.at[pl.ds(r * 2 * sn, 2 * sn)]], buf, sem
            )

        def write(r, half, buf, sem):
            t = r * 2 + half
            return pltpu.make_async_copy(
                buf.at[pl.ds(half * sn, sn)],
                out_hbm.at[pl.ds(sbase, sn), t, pl.ds(0, D)],
                sem,
            )

        R = T // 2
        gather(0, rows0, g0).start()
        gather(1, rows1, g1).start()
        for gr in range(R // 2):
            for b, (buf, gsem, wsem) in enumerate(
                ((rows0, g0, w0), (rows1, g1, w1))
            ):
                r = gr * 2 + b
                gather(r, buf, gsem).wait()
                write(r, 0, buf, wsem).start()
                write(r, 1, buf, wsem).start()
                write(r, 0, buf, wsem).wait()
                write(r, 1, buf, wsem).wait()
                if r + 2 < R:
                    gather(r + 2, buf, gsem).start()

    return gather_kernel


def kernel(tokens, table):
    nb, t = tokens.shape
    d = table.shape[1]
    tp = (t + 7) // 8 * 8
    dp = 128
    idx_tmajor = tokens.T.reshape(nb * t)
    out_padded = _make_sc_gather(nb, t, d, tp, dp)(idx_tmajor, table)
    return out_padded[:, :t, :d]
